# Initial kernel scaffold; baseline (speedup 1.0000x reference)
import jax
import jax.numpy as jnp
from jax.experimental import pallas as pl


def kernel(input_embd, edge_index, W1, b1, W2, b2, W3, b3):
    raise NotImplementedError("write your pallas kernel here")


# speedup vs baseline: 23.9839x; 23.9839x over previous
import functools

import jax
import jax.numpy as jnp
from jax import lax
from jax.experimental import pallas as pl
from jax.experimental.pallas import tpu as pltpu
from jax.experimental.pallas import tpu_sc as plsc

N = 10000
E = 320000
D_IN = 128
D_HID = 512

NC = 2
NS = 16
NT = NC * NS
EPT = E // NT
CH = 128
FULL = EPT // CH
TAIL = EPT - FULL * CH
STRIPE = (N // NS) // 8 * 8
REM = N - NS * STRIPE

_mesh = plsc.VectorSubcoreMesh(core_axis_name="c", subcore_axis_name="s")


@functools.partial(
    pl.kernel,
    out_type=jax.ShapeDtypeStruct((NC, N, 16), jnp.float32),
    mesh=_mesh,
    scratch_types=[
        pltpu.VMEM((CH, 16), jnp.float32),
        pltpu.VMEM((CH,), jnp.int32),
        pltpu.VMEM((TAIL,), jnp.int32),
        pltpu.VMEM_SHARED((N, 16), jnp.float32),
    ],
)
def _degree_kernel(edge_hbm, zeros16_hbm, deg_hbm, ones_v, dst_v, dst_t, deg_sh):
    c = lax.axis_index("c")
    s = lax.axis_index("s")

    def init_ones(r, carry):
        ones_v[r, :] = jnp.ones((16,), jnp.float32)
        return carry

    lax.fori_loop(0, CH, init_ones, 0)

    pltpu.sync_copy(zeros16_hbm.at[pl.ds(s * STRIPE, STRIPE)],
                    deg_sh.at[pl.ds(s * STRIPE, STRIPE)])

    @pl.when(s == NS - 1)
    def _():
        pltpu.sync_copy(zeros16_hbm.at[pl.ds(NS * STRIPE, REM)],
                        deg_sh.at[pl.ds(NS * STRIPE, REM)])

    plsc.subcore_barrier()

    base_e = E + (c * NS + s) * EPT

    def chunk(k, carry):
        pltpu.sync_copy(edge_hbm.at[pl.ds(base_e + k * CH, CH)], dst_v)
        pltpu.sync_copy(ones_v, deg_sh.at[dst_v], add=True)
        return carry

    lax.fori_loop(0, FULL, chunk, 0)
    pltpu.sync_copy(edge_hbm.at[pl.ds(base_e + FULL * CH, TAIL)], dst_t)
    pltpu.sync_copy(ones_v.at[pl.ds(0, TAIL)], deg_sh.at[dst_t], add=True)

    plsc.subcore_barrier()
    pltpu.sync_copy(deg_sh.at[pl.ds(s * STRIPE, STRIPE)],
                    deg_hbm.at[c, pl.ds(s * STRIPE, STRIPE)])

    @pl.when(s == NS - 1)
    def _():
        pltpu.sync_copy(deg_sh.at[pl.ds(NS * STRIPE, REM)],
                        deg_hbm.at[c, pl.ds(NS * STRIPE, REM)])


@functools.partial(
    pl.kernel,
    out_type=jax.ShapeDtypeStruct((NC, N, D_IN), jnp.float32),
    mesh=_mesh,
    scratch_types=[
        pltpu.VMEM((CH,), jnp.int32),
        pltpu.VMEM((CH,), jnp.int32),
        pltpu.VMEM((CH, D_IN), jnp.float32),
        pltpu.VMEM((TAIL,), jnp.int32),
        pltpu.VMEM((TAIL,), jnp.int32),
        pltpu.VMEM((TAIL, D_IN), jnp.float32),
        pltpu.VMEM_SHARED((N, D_IN), jnp.float32),
        pltpu.SemaphoreType.DMA,
    ],
)
def _scatter_kernel(edge_hbm, y_hbm, zeros_hbm, out_hbm,
                    src_v, dst_v, rows_v, src_t, dst_t, rows_t, acc_sh, sem):
    c = lax.axis_index("c")
    s = lax.axis_index("s")

    pltpu.sync_copy(zeros_hbm.at[pl.ds(s * STRIPE, STRIPE)],
                    acc_sh.at[pl.ds(s * STRIPE, STRIPE)])

    @pl.when(s == NS - 1)
    def _():
        pltpu.sync_copy(zeros_hbm.at[pl.ds(NS * STRIPE, REM)],
                        acc_sh.at[pl.ds(NS * STRIPE, REM)])

    plsc.subcore_barrier()

    base_e = (c * NS + s) * EPT

    def chunk(k, carry):
        b = base_e + k * CH
        pltpu.sync_copy(edge_hbm.at[pl.ds(b, CH)], src_v)
        pltpu.sync_copy(edge_hbm.at[pl.ds(E + b, CH)], dst_v)
        pltpu.async_copy(y_hbm.at[src_v], rows_v, sem).wait()
        pltpu.sync_copy(rows_v, acc_sh.at[dst_v], add=True)
        return carry

    lax.fori_loop(0, FULL, chunk, 0)

    bt = base_e + FULL * CH
    pltpu.sync_copy(edge_hbm.at[pl.ds(bt, TAIL)], src_t)
    pltpu.sync_copy(edge_hbm.at[pl.ds(E + bt, TAIL)], dst_t)
    pltpu.async_copy(y_hbm.at[src_t], rows_t, sem).wait()
    pltpu.sync_copy(rows_t, acc_sh.at[dst_t], add=True)

    plsc.subcore_barrier()
    pltpu.sync_copy(acc_sh.at[pl.ds(s * STRIPE, STRIPE)],
                    out_hbm.at[c, pl.ds(s * STRIPE, STRIPE)])

    @pl.when(s == NS - 1)
    def _():
        pltpu.sync_copy(acc_sh.at[pl.ds(NS * STRIPE, REM)],
                        out_hbm.at[c, pl.ds(NS * STRIPE, REM)])


_RB = 1000


def _scale_body(deg_ref, x_ref, y_ref):
    d16 = deg_ref[0] + deg_ref[1]
    deg = jnp.sum(d16, axis=1) * (1.0 / 16.0) + 1.0
    dinv = lax.rsqrt(deg)
    y_ref[...] = x_ref[...] * dinv[:, None]


def _scale(deg16, x):
    return pl.pallas_call(
        _scale_body,
        grid=(N // _RB,),
        in_specs=[
            pl.BlockSpec((NC, _RB, 16), lambda i: (0, i, 0)),
            pl.BlockSpec((_RB, D_IN), lambda i: (i, 0)),
        ],
        out_specs=pl.BlockSpec((_RB, D_IN), lambda i: (i, 0)),
        out_shape=jax.ShapeDtypeStruct((N, D_IN), jnp.float32),
    )(deg16, x)


def _mlp_body(p_ref, y_ref, deg_ref, w1_ref, b1_ref, w2_ref, b2_ref,
              w3_ref, b3_ref, out_ref):
    d16 = deg_ref[0] + deg_ref[1]
    deg = jnp.sum(d16, axis=1) * (1.0 / 16.0) + 1.0
    dinv = lax.rsqrt(deg)
    agg = (p_ref[0] + p_ref[1] + y_ref[...]) * dinv[:, None]
    h = jnp.dot(agg, w1_ref[...], preferred_element_type=jnp.float32,
                precision=lax.Precision.HIGHEST) + b1_ref[...]
    h = jnp.where(h > 0, h, 0.2 * h)
    h = jnp.dot(h, w2_ref[...], preferred_element_type=jnp.float32,
                precision=lax.Precision.HIGHEST) + b2_ref[...]
    h = jnp.where(h > 0, h, 0.2 * h)
    out_ref[...] = jnp.dot(h, w3_ref[...], preferred_element_type=jnp.float32,
                           precision=lax.Precision.HIGHEST) + b3_ref[...]


def _mlp(parts, y, deg16, W1, b1, W2, b2, W3, b3):
    return pl.pallas_call(
        _mlp_body,
        grid=(N // _RB,),
        in_specs=[
            pl.BlockSpec((NC, _RB, D_IN), lambda i: (0, i, 0)),
            pl.BlockSpec((_RB, D_IN), lambda i: (i, 0)),
            pl.BlockSpec((NC, _RB, 16), lambda i: (0, i, 0)),
            pl.BlockSpec((D_IN, D_HID), lambda i: (0, 0)),
            pl.BlockSpec((D_HID,), lambda i: (0,)),
            pl.BlockSpec((D_HID, D_HID), lambda i: (0, 0)),
            pl.BlockSpec((D_HID,), lambda i: (0,)),
            pl.BlockSpec((D_HID, 1), lambda i: (0, 0)),
            pl.BlockSpec((1,), lambda i: (0,)),
        ],
        out_specs=pl.BlockSpec((_RB, 1), lambda i: (i, 0)),
        out_shape=jax.ShapeDtypeStruct((N, 1), jnp.float32),
    )(parts, y, deg16, W1, b1, W2, b2, W3, b3)


def kernel(input_embd, edge_index, W1, b1, W2, b2, W3, b3):
    edge_flat = edge_index.reshape(-1)
    zeros16 = jnp.zeros((N, 16), jnp.float32)
    zeros128 = jnp.zeros((N, D_IN), jnp.float32)
    deg16 = _degree_kernel(edge_flat, zeros16)
    y = _scale(deg16, input_embd)
    parts = _scatter_kernel(edge_flat, y, zeros128)
    return _mlp(parts, y, deg16, W1, b1, W2, b2, W3, b3)
